# Initial kernel scaffold; baseline (speedup 1.0000x reference)
import jax
import jax.numpy as jnp
from jax.experimental import pallas as pl


def kernel(feature_data, edge_info, W1, b1, W2, b2):
    raise NotImplementedError("write your pallas kernel here")


# speedup vs baseline: 6.2382x; 6.2382x over previous
import functools

import jax
import jax.numpy as jnp
from jax import lax
from jax.experimental import pallas as pl
from jax.experimental.pallas import tpu as pltpu
from jax.experimental.pallas import tpu_sc as plsc

N = 10000
F0, F1, F2 = 128, 256, 128
E = 320000
NC, NS = 2, 16
NW = NC * NS
G = 128
EPAD = 327680
NPAD = 10240
RPT = NPAD // NS
R = 640
R2 = 80

_f32 = jnp.float32


@functools.cache
def _sc_kernels():
    mesh = plsc.VectorSubcoreMesh(core_axis_name="c", subcore_axis_name="s")

    k1 = functools.partial(
        pl.kernel,
        out_type=jax.ShapeDtypeStruct((2 * NPAD, 16), _f32),
        mesh=mesh,
        scratch_types=[
            pltpu.VMEM((G,), jnp.int32),
            pltpu.VMEM((G, 16), _f32),
            pltpu.VMEM_SHARED((NPAD, 16), _f32),
        ],
        name="gcn_sc_hist",
    )(_k1_hist)
    k3 = functools.partial(
        pl.kernel,
        out_type=jax.ShapeDtypeStruct((2 * NPAD, F0), _f32),
        mesh=mesh,
        scratch_types=[
            pltpu.VMEM((G,), jnp.int32),
            pltpu.VMEM((G,), jnp.int32),
            pltpu.VMEM((G, F0), _f32),
            pltpu.VMEM_SHARED((NPAD, F0), _f32),
            pltpu.SemaphoreType.DMA,
        ],
        name="gcn_sc_spmm1",
    )(_k3_spmm1)
    k5 = functools.partial(
        pl.kernel,
        out_type=jax.ShapeDtypeStruct((2 * NPAD, F2), _f32),
        mesh=mesh,
        scratch_types=[
            pltpu.VMEM((G,), jnp.int32),
            pltpu.VMEM((G,), jnp.int32),
            pltpu.VMEM((G, F2), _f32),
            pltpu.VMEM_SHARED((NPAD, F2), _f32),
            pltpu.SemaphoreType.DMA,
        ],
        name="gcn_sc_spmm2",
    )(_k5_spmm2)
    return k1, k3, k5


def _k1_hist(dst_hbm, ones_hbm, z16_hbm, out, didx, ones_v, acc):
    c = lax.axis_index("c")
    s = lax.axis_index("s")
    pltpu.sync_copy(z16_hbm, acc.at[pl.ds(s * RPT, RPT)])
    pltpu.sync_copy(ones_hbm, ones_v)
    plsc.subcore_barrier()
    base = (c * NS + s) * (EPAD // NW)

    def body(g, carry):
        off = pl.multiple_of(base + g * G, G)
        pltpu.sync_copy(dst_hbm.at[pl.ds(off, G)], didx)
        pltpu.sync_copy(ones_v, acc.at[didx], add=True)
        return carry

    lax.fori_loop(0, EPAD // NW // G, body, 0)
    plsc.subcore_barrier()
    pltpu.sync_copy(acc.at[pl.ds(s * RPT, RPT)],
                    out.at[pl.ds(c * NPAD + s * RPT, RPT)])


def _k2_body(x_ref, w1_ref, d0_ref, d1_ref, dinv_ref, ycat_ref):
    deg = d0_ref[0][:, :1] + d1_ref[0][:, :1] + 1.0
    dinv = lax.rsqrt(deg)
    xw = jnp.dot(x_ref[...], w1_ref[...], preferred_element_type=_f32)
    ycat_ref[...] = xw * dinv
    dinv_ref[...] = jnp.broadcast_to(dinv, (R, F0))


_NB = NPAD // R


def _k2_prep(x_p, w1, d0, d1):
    return pl.pallas_call(
        _k2_body,
        grid=(_NB, 2),
        in_specs=[
            pl.BlockSpec((R, F0), lambda j, h: (j, 0)),
            pl.BlockSpec((F0, F0), lambda j, h: (0, h)),
            pl.BlockSpec((1, R, 16), lambda j, h: (0, j, 0)),
            pl.BlockSpec((1, R, 16), lambda j, h: (1, j, 0)),
        ],
        out_specs=[
            pl.BlockSpec((R, F0), lambda j, h: (j, 0)),
            pl.BlockSpec((R, F0), lambda j, h: (h * _NB + j, 0)),
        ],
        out_shape=[
            jax.ShapeDtypeStruct((NPAD, F0), _f32),
            jax.ShapeDtypeStruct((2 * NPAD, F0), _f32),
        ],
    )(x_p, w1, d0, d1)


def _k3_spmm1(src2_hbm, dst_hbm, ycat_hbm, out,
              sidx, didx, rows, acc, sem):
    c = lax.axis_index("c")
    s = lax.axis_index("s")
    pltpu.sync_copy(ycat_hbm.at[pl.ds(c * NPAD + s * RPT, RPT)],
                    acc.at[pl.ds(s * RPT, RPT)])
    plsc.subcore_barrier()
    base = s * (EPAD // NS)

    def body(g, carry):
        off = pl.multiple_of(base + g * G, G)
        pltpu.sync_copy(src2_hbm.at[pl.ds(c * EPAD + off, G)], sidx)
        pltpu.sync_copy(dst_hbm.at[pl.ds(off, G)], didx)
        pltpu.async_copy(ycat_hbm.at[sidx], rows, sem).wait()
        pltpu.sync_copy(rows, acc.at[didx], add=True)
        return carry

    lax.fori_loop(0, EPAD // NS // G, body, 0)
    plsc.subcore_barrier()
    pltpu.sync_copy(acc.at[pl.ds(s * RPT, RPT)],
                    out.at[pl.ds(c * NPAD + s * RPT, RPT)])


def _k4_body(slo_ref, shi_ref, dinv_ref, b1_ref, w2_ref, y2_ref):
    dv = dinv_ref[...]
    b1 = b1_ref[...]
    h0 = jnp.maximum(slo_ref[0] * dv + b1[:, :F0], 0.0)
    h1 = jnp.maximum(shi_ref[0] * dv + b1[:, F0:], 0.0)
    w2 = w2_ref[...]
    z = (jnp.dot(h0, w2[:F0, :], preferred_element_type=_f32)
         + jnp.dot(h1, w2[F0:, :], preferred_element_type=_f32))
    rows = pl.program_id(0) * R + lax.broadcasted_iota(jnp.int32, (R, F2), 0)
    keep = (rows < N) & (pl.program_id(1) == 0)
    y2_ref[...] = jnp.where(keep, z * dv, 0.0)


def _k4_mid(s3, dinv, b1, w2):
    return pl.pallas_call(
        _k4_body,
        grid=(NPAD // R, 2),
        in_specs=[
            pl.BlockSpec((1, R, F0), lambda j, h: (0, j, 0)),
            pl.BlockSpec((1, R, F0), lambda j, h: (1, j, 0)),
            pl.BlockSpec((R, F0), lambda j, h: (j, 0)),
            pl.BlockSpec((1, F1), lambda j, h: (0, 0)),
            pl.BlockSpec((F1, F2), lambda j, h: (0, 0)),
        ],
        out_specs=pl.BlockSpec((R, F2), lambda j, h: (h * (NPAD // R) + j, 0)),
        out_shape=jax.ShapeDtypeStruct((2 * NPAD, F2), _f32),
    )(s3, s3, dinv, b1, w2)


def _k5_spmm2(src_hbm, dst_hbm, y2z_hbm, out,
              sidx, didx, rows, acc, sem):
    c = lax.axis_index("c")
    s = lax.axis_index("s")
    pltpu.sync_copy(y2z_hbm.at[pl.ds(c * NPAD + s * RPT, RPT)],
                    acc.at[pl.ds(s * RPT, RPT)])
    plsc.subcore_barrier()
    base = (c * NS + s) * (EPAD // NW)

    def body(g, carry):
        off = pl.multiple_of(base + g * G, G)
        pltpu.sync_copy(src_hbm.at[pl.ds(off, G)], sidx)
        pltpu.sync_copy(dst_hbm.at[pl.ds(off, G)], didx)
        pltpu.async_copy(y2z_hbm.at[sidx], rows, sem).wait()
        pltpu.sync_copy(rows, acc.at[didx], add=True)
        return carry

    lax.fori_loop(0, EPAD // NW // G, body, 0)
    plsc.subcore_barrier()
    pltpu.sync_copy(acc.at[pl.ds(s * RPT, RPT)],
                    out.at[pl.ds(c * NPAD + s * RPT, RPT)])


def _k6_body(p0_ref, p1_ref, dinv_ref, b2_ref, out_ref):
    z = (p0_ref[...] + p1_ref[...]) * dinv_ref[...][:, :F2] + b2_ref[...]
    zm = z - jnp.max(z, axis=1, keepdims=True)
    lse = jnp.log(jnp.sum(jnp.exp(zm), axis=1, keepdims=True))
    out_ref[...] = zm - lse


def _k6_final(pcat, dinv, b2):
    return pl.pallas_call(
        _k6_body,
        grid=(N // R2,),
        in_specs=[
            pl.BlockSpec((R2, F2), lambda j: (j, 0)),
            pl.BlockSpec((R2, F2), lambda j: (NPAD // R2 + j, 0)),
            pl.BlockSpec((R2, F0), lambda j: (j, 0)),
            pl.BlockSpec((1, F2), lambda j: (0, 0)),
        ],
        out_specs=pl.BlockSpec((R2, F2), lambda j: (j, 0)),
        out_shape=jax.ShapeDtypeStruct((N, F2), _f32),
    )(pcat, pcat, dinv, b2)


def kernel(feature_data, edge_info, W1, b1, W2, b2):
    src = edge_info[0].astype(jnp.int32)
    dst = edge_info[1].astype(jnp.int32)
    pad = jnp.full((EPAD - E,), N, dtype=jnp.int32)
    src_p = jnp.concatenate([src, pad])
    dst_p = jnp.concatenate([dst, pad])
    x_p = jnp.pad(feature_data, ((0, NPAD - N), (0, 0)))
    ones16 = jnp.ones((G, 16), dtype=_f32)
    z16 = jnp.zeros((RPT, 16), dtype=_f32)

    src2 = jnp.concatenate([src_p, src_p + NPAD])
    k1, k3, k5 = _sc_kernels()
    dcat = k1(dst_p, ones16, z16)
    d3 = dcat.reshape(2, NPAD, 16)
    dinv, ycat = _k2_prep(x_p, W1, d3, d3)
    scat = k3(src2, dst_p, ycat)
    s3 = scat.reshape(2, NPAD, F0)
    y2z = _k4_mid(s3, dinv, b1.reshape(1, F1), W2)
    pcat = k5(src_p, dst_p, y2z)
    return _k6_final(pcat, dinv, b2.reshape(1, F2))
